# baseline (speedup 1.0000x reference)
import functools

import jax
import jax.numpy as jnp
from jax import lax
from jax.experimental import pallas as pl
from jax.experimental.pallas import tpu as pltpu
from jax.experimental.pallas import tpu_sc as plsc

NC = 2
NS = 16
L = 16
CHUNK = 128
GRP = 8
ORDER = 8


def _prop_sc(n_pad, f, e_pad, x_pad, src2, dst2, w2):
    fh = f // NC
    rows_per_tile = n_pad // NS
    chunks_per_tile = (e_pad // CHUNK) // NS
    n_groups = chunks_per_tile // GRP
    n_rb = rows_per_tile // CHUNK
    mesh = plsc.VectorSubcoreMesh(
        core_axis_name="c", subcore_axis_name="s", num_cores=NC, num_subcores=NS
    )

    @functools.partial(
        pl.kernel,
        out_type=jax.ShapeDtypeStruct((ORDER, n_pad, f), jnp.float32),
        mesh=mesh,
        compiler_params=pltpu.CompilerParams(
            use_tc_tiling_on_sc=False, needs_layout_passes=False
        ),
        scratch_types=[
            pltpu.VMEM_SHARED((2 * n_pad, fh), jnp.float32),
            pltpu.VMEM((CHUNK, fh), jnp.float32),
            pltpu.VMEM((CHUNK, fh), jnp.float32),
            pltpu.VMEM((CHUNK, fh), jnp.float32),
            pltpu.VMEM((CHUNK, fh), jnp.float32),
            pltpu.VMEM((GRP, CHUNK), jnp.int32),
            pltpu.VMEM((GRP, CHUNK), jnp.int32),
            pltpu.VMEM((GRP, CHUNK), jnp.float32),
            pltpu.VMEM((GRP, CHUNK), jnp.int32),
            pltpu.VMEM((GRP, CHUNK), jnp.int32),
            pltpu.VMEM((GRP, CHUNK), jnp.float32),
            pltpu.SemaphoreType.DMA,
            pltpu.SemaphoreType.DMA,
            pltpu.SemaphoreType.DMA,
            pltpu.SemaphoreType.DMA,
            pltpu.SemaphoreType.DMA,
            pltpu.SemaphoreType.DMA,
            pltpu.SemaphoreType.DMA,
            pltpu.SemaphoreType.DMA,
            pltpu.SemaphoreType.DMA,
            pltpu.SemaphoreType.DMA,
        ],
    )
    def prop_kernel(x_hbm, src_hbm, dst_hbm, w_hbm, xs_hbm,
                    xbuf, rows_0, rows_1, rows_2, rows_3,
                    sidx0, didx0, wbuf0, sidx1, didx1, wbuf1,
                    isem0, isem1, gsem_0, gsem_1, gsem_2, gsem_3,
                    ssem_0, ssem_1, ssem_2, ssem_3):
        c = lax.axis_index("c")
        s = lax.axis_index("s")
        fbase = c * fh
        r0 = s * rows_per_tile
        row_slice = pl.ds(r0, rows_per_tile)
        fcol = pl.ds(fbase, fh)

        def zero_rows_0():
            @pl.loop(0, CHUNK)
            def _(i):
                for j in range(fh // L):
                    rows_0[i, pl.ds(j * L, L)] = jnp.zeros((L,), jnp.float32)

        pltpu.sync_copy(x_hbm.at[row_slice, fcol], xbuf.at[row_slice])
        zero_rows_0()

        @pl.loop(0, n_rb)
        def _(b):
            pltpu.sync_copy(rows_0, xbuf.at[pl.ds(n_pad + r0 + b * CHUNK, CHUNK)])

        plsc.subcore_barrier()

        cbase = s * chunks_per_tile
        rows = (rows_0, rows_1, rows_2, rows_3)
        gsems = (gsem_0, gsem_1, gsem_2, gsem_3)
        ssems = (ssem_0, ssem_1, ssem_2, ssem_3)
        idx0 = (sidx0, didx0, wbuf0, isem0)
        idx1 = (sidx1, didx1, wbuf1, isem1)
        last_row = cbase + (n_groups - 1) * GRP

        def load_idx(g_row, bufs):
            sb, db, wb, sem = bufs
            pltpu.async_copy(src_hbm.at[pl.ds(g_row, GRP)], sb, sem)
            pltpu.async_copy(dst_hbm.at[pl.ds(g_row, GRP)], db, sem)
            pltpu.async_copy(w_hbm.at[pl.ds(g_row, GRP)], wb, sem)

        def wait_idx(bufs, src_off, dst_off):
            sb, db, wb, sem = bufs
            pltpu.make_async_copy(src_hbm.at[pl.ds(0, GRP)], sb, sem).wait()
            pltpu.make_async_copy(dst_hbm.at[pl.ds(0, GRP)], db, sem).wait()
            pltpu.make_async_copy(w_hbm.at[pl.ds(0, GRP)], wb, sem).wait()
            so = jnp.full((L,), src_off, jnp.int32)
            do = jnp.full((L,), dst_off, jnp.int32)

            @pl.loop(0, GRP)
            def _(kk):
                for jj in range(CHUNK // L):
                    sl = pl.ds(jj * L, L)
                    sb[kk, sl] = sb[kk, sl] + so
                    db[kk, sl] = db[kk, sl] + do

        def process_group(bufs):
            sb, db, wb, _ = bufs
            gathers = [None] * GRP
            scats = [None] * GRP
            for k in range(2):
                gathers[k] = pltpu.async_copy(
                    xbuf.at[sb.at[k]], rows[k], gsems[k])
            for k in range(GRP):
                q = k % 4
                cur = rows[q]
                gathers[k].wait()
                if k + 2 < GRP:
                    if k >= 2:
                        scats[k - 2].wait()
                    gathers[k + 2] = pltpu.async_copy(
                        xbuf.at[sb.at[k + 2]], rows[(k + 2) % 4],
                        gsems[(k + 2) % 4])

                @pl.loop(0, CHUNK, step=L, unroll=2)
                def _(i0):
                    wv16 = wb[k, pl.ds(i0, L)]
                    for ii in range(L):
                        wv = jnp.full((L,), wv16[ii], jnp.float32)
                        for j in range(fh // L):
                            fs = pl.ds(j * L, L)
                            cur[i0 + ii, fs] = cur[i0 + ii, fs] * wv

                scats[k] = pltpu.async_copy(
                    cur, xbuf.at[db.at[k]], ssems[q], add=True)

            for k in range(GRP - 4, GRP):
                scats[k].wait()

        @pl.loop(0, ORDER)
        def _(r):
            ps = lax.rem(r, 2)
            src_off = ps * n_pad
            dst_off = (1 - ps) * n_pad

            load_idx(cbase, idx0)
            load_idx(cbase + GRP, idx1)

            @pl.loop(0, n_groups // 2)
            def _(t):
                g0row = cbase + (2 * t) * GRP
                wait_idx(idx0, src_off, dst_off)
                process_group(idx0)
                load_idx(jnp.minimum(g0row + 2 * GRP, last_row), idx0)
                wait_idx(idx1, src_off, dst_off)
                process_group(idx1)
                load_idx(jnp.minimum(g0row + 3 * GRP, last_row), idx1)

            wait_idx(idx0, 0, 0)
            wait_idx(idx1, 0, 0)
            plsc.subcore_barrier()

            pltpu.sync_copy(xbuf.at[pl.ds(dst_off + r0, rows_per_tile)],
                            xs_hbm.at[r, row_slice, fcol])
            zero_rows_0()

            @pl.loop(0, n_rb)
            def _(b):
                pltpu.sync_copy(
                    rows_0, xbuf.at[pl.ds(src_off + r0 + b * CHUNK, CHUNK)])

            plsc.subcore_barrier()

    return prop_kernel(x_pad, src2, dst2, w2)


def _head_tc(x_pad, xs, w1, b1, w2, b2):
    n_pad, f = x_pad.shape
    order = xs.shape[0]
    hid = w1.shape[1]
    c_out = w2.shape[1]
    br = 1280

    def head_kernel(x_ref, xs_ref, w1_ref, b1_ref, w2_ref, b2_ref, o_ref):
        x = x_ref[...] + jnp.sum(xs_ref[...], axis=0)
        nrm = jnp.sqrt(jnp.sum(x * x, axis=1, keepdims=True))
        x = x / (1e-12 + nrm)
        h = jnp.dot(x, w1_ref[...], preferred_element_type=jnp.float32)
        h = h + b1_ref[...]
        h = jnp.maximum(h, 0.0)
        hn = jnp.sqrt(jnp.sum(h * h, axis=1, keepdims=True))
        h = h / (1e-12 + hn)
        o = jnp.dot(h, w2_ref[...], preferred_element_type=jnp.float32)
        o_ref[...] = o + b2_ref[...]

    return pl.pallas_call(
        head_kernel,
        grid=(n_pad // br,),
        in_specs=[
            pl.BlockSpec((br, f), lambda i: (i, 0)),
            pl.BlockSpec((order, br, f), lambda i: (0, i, 0)),
            pl.BlockSpec((f, hid), lambda i: (0, 0)),
            pl.BlockSpec((1, hid), lambda i: (0, 0)),
            pl.BlockSpec((hid, c_out), lambda i: (0, 0)),
            pl.BlockSpec((1, c_out), lambda i: (0, 0)),
        ],
        out_specs=pl.BlockSpec((br, c_out), lambda i: (i, 0)),
        out_shape=jax.ShapeDtypeStruct((n_pad, c_out), jnp.float32),
    )(x_pad, xs, w1, b1, w2, b2)


def kernel(X, edge_index, edge_weight, W1, b1, W2, b2):
    n, f = X.shape
    e = edge_weight.shape[0]
    n_pad = ((n + NS * L - 1) // (NS * L)) * (NS * L)
    step = NS * GRP * CHUNK
    e_pad = ((e + step - 1) // step) * step

    src = edge_index[0].astype(jnp.int32)
    dst = edge_index[1].astype(jnp.int32)
    w = edge_weight.astype(jnp.float32)
    if e_pad != e:
        pad = e_pad - e
        src = jnp.concatenate([src, jnp.zeros((pad,), jnp.int32)])
        dst = jnp.concatenate([dst, jnp.zeros((pad,), jnp.int32)])
        w = jnp.concatenate([w, jnp.zeros((pad,), jnp.float32)])
    src2 = src.reshape(e_pad // CHUNK, CHUNK)
    dst2 = dst.reshape(e_pad // CHUNK, CHUNK)
    w2 = w.reshape(e_pad // CHUNK, CHUNK)
    x_pad = X if n_pad == n else jnp.pad(X, ((0, n_pad - n), (0, 0)))

    xs = _prop_sc(n_pad, f, e_pad, x_pad, src2, dst2, w2)
    out = _head_tc(x_pad, xs, W1, b1.reshape(1, -1), W2, b2.reshape(1, -1))
    return out[:n]


# speedup vs baseline: 1.0029x; 1.0029x over previous
import functools

import jax
import jax.numpy as jnp
from jax import lax
from jax.experimental import pallas as pl
from jax.experimental.pallas import tpu as pltpu
from jax.experimental.pallas import tpu_sc as plsc

NC = 2
NS = 16
L = 16
CHUNK = 128
GRP = 8
ORDER = 8


def _prop_sc(n_pad, f, e_pad, x_pad, src2, dst2, w2):
    fh = f // NC
    rows_per_tile = n_pad // NS
    chunks_per_tile = (e_pad // CHUNK) // NS
    n_groups = chunks_per_tile // GRP
    n_rb = rows_per_tile // CHUNK
    mesh = plsc.VectorSubcoreMesh(
        core_axis_name="c", subcore_axis_name="s", num_cores=NC, num_subcores=NS
    )

    @functools.partial(
        pl.kernel,
        out_type=jax.ShapeDtypeStruct((ORDER, n_pad, f), jnp.float32),
        mesh=mesh,
        compiler_params=pltpu.CompilerParams(
            use_tc_tiling_on_sc=False, needs_layout_passes=False
        ),
        scratch_types=[
            pltpu.VMEM_SHARED((2 * n_pad, fh), jnp.float32),
            pltpu.VMEM((CHUNK, fh), jnp.float32),
            pltpu.VMEM((CHUNK, fh), jnp.float32),
            pltpu.VMEM((CHUNK, fh), jnp.float32),
            pltpu.VMEM((CHUNK, fh), jnp.float32),
            pltpu.VMEM((GRP, CHUNK), jnp.int32),
            pltpu.VMEM((GRP, CHUNK), jnp.int32),
            pltpu.VMEM((GRP, CHUNK), jnp.float32),
            pltpu.VMEM((GRP, CHUNK), jnp.int32),
            pltpu.VMEM((GRP, CHUNK), jnp.int32),
            pltpu.VMEM((GRP, CHUNK), jnp.float32),
            pltpu.SemaphoreType.DMA,
            pltpu.SemaphoreType.DMA,
            pltpu.SemaphoreType.DMA,
            pltpu.SemaphoreType.DMA,
            pltpu.SemaphoreType.DMA,
            pltpu.SemaphoreType.DMA,
            pltpu.SemaphoreType.DMA,
            pltpu.SemaphoreType.DMA,
            pltpu.SemaphoreType.DMA,
            pltpu.SemaphoreType.DMA,
        ],
    )
    def prop_kernel(x_hbm, src_hbm, dst_hbm, w_hbm, xs_hbm,
                    xbuf, rows_0, rows_1, rows_2, rows_3,
                    sidx0, didx0, wbuf0, sidx1, didx1, wbuf1,
                    isem0, isem1, gsem_0, gsem_1, gsem_2, gsem_3,
                    ssem_0, ssem_1, ssem_2, ssem_3):
        c = lax.axis_index("c")
        s = lax.axis_index("s")
        fbase = c * fh
        r0 = s * rows_per_tile
        row_slice = pl.ds(r0, rows_per_tile)
        fcol = pl.ds(fbase, fh)

        def zero_rows_0():
            @pl.loop(0, CHUNK)
            def _(i):
                for j in range(fh // L):
                    rows_0[i, pl.ds(j * L, L)] = jnp.zeros((L,), jnp.float32)

        pltpu.sync_copy(x_hbm.at[row_slice, fcol], xbuf.at[row_slice])
        zero_rows_0()

        @pl.loop(0, n_rb)
        def _(b):
            pltpu.sync_copy(rows_0, xbuf.at[pl.ds(n_pad + r0 + b * CHUNK, CHUNK)])

        plsc.subcore_barrier()

        cbase = s * chunks_per_tile
        rows = (rows_0, rows_1, rows_2, rows_3)
        gsems = (gsem_0, gsem_1, gsem_2, gsem_3)
        ssems = (ssem_0, ssem_1, ssem_2, ssem_3)
        idx0 = (sidx0, didx0, wbuf0, isem0)
        idx1 = (sidx1, didx1, wbuf1, isem1)
        last_row = cbase + (n_groups - 1) * GRP

        def load_idx(g_row, bufs):
            sb, db, wb, sem = bufs
            pltpu.async_copy(src_hbm.at[pl.ds(g_row, GRP)], sb, sem)
            pltpu.async_copy(dst_hbm.at[pl.ds(g_row, GRP)], db, sem)
            pltpu.async_copy(w_hbm.at[pl.ds(g_row, GRP)], wb, sem)

        def wait_idx(bufs, src_off, dst_off):
            sb, db, wb, sem = bufs
            pltpu.make_async_copy(src_hbm.at[pl.ds(0, GRP)], sb, sem).wait()
            pltpu.make_async_copy(dst_hbm.at[pl.ds(0, GRP)], db, sem).wait()
            pltpu.make_async_copy(w_hbm.at[pl.ds(0, GRP)], wb, sem).wait()
            so = jnp.full((L,), src_off, jnp.int32)
            do = jnp.full((L,), dst_off, jnp.int32)

            @pl.loop(0, GRP)
            def _(kk):
                for jj in range(CHUNK // L):
                    sl = pl.ds(jj * L, L)
                    sb[kk, sl] = sb[kk, sl] + so
                    db[kk, sl] = db[kk, sl] + do

        def process_group(bufs):
            sb, db, wb, _ = bufs
            gathers = [None] * GRP
            scats = [None] * GRP
            for k in range(2):
                gathers[k] = pltpu.async_copy(
                    xbuf.at[sb.at[k]], rows[k], gsems[k])
            for k in range(GRP):
                q = k % 4
                cur = rows[q]
                gathers[k].wait()
                if k + 2 < GRP:
                    if k >= 2:
                        scats[k - 2].wait()
                    gathers[k + 2] = pltpu.async_copy(
                        xbuf.at[sb.at[k + 2]], rows[(k + 2) % 4],
                        gsems[(k + 2) % 4])

                @pl.loop(0, CHUNK, step=L)
                def _(i0):
                    wv16 = wb[k, pl.ds(i0, L)]
                    for ii in range(L):
                        wv = jnp.full((L,), wv16[ii], jnp.float32)
                        for j in range(fh // L):
                            fs = pl.ds(j * L, L)
                            cur[i0 + ii, fs] = cur[i0 + ii, fs] * wv

                scats[k] = pltpu.async_copy(
                    cur, xbuf.at[db.at[k]], ssems[q], add=True)

            for k in range(GRP - 4, GRP):
                scats[k].wait()

        @pl.loop(0, ORDER)
        def _(r):
            ps = lax.rem(r, 2)
            src_off = ps * n_pad
            dst_off = (1 - ps) * n_pad

            load_idx(cbase, idx0)
            load_idx(cbase + GRP, idx1)

            @pl.loop(0, n_groups // 2)
            def _(t):
                g0row = cbase + (2 * t) * GRP
                wait_idx(idx0, src_off, dst_off)
                process_group(idx0)
                load_idx(jnp.minimum(g0row + 2 * GRP, last_row), idx0)
                wait_idx(idx1, src_off, dst_off)
                process_group(idx1)
                load_idx(jnp.minimum(g0row + 3 * GRP, last_row), idx1)

            wait_idx(idx0, 0, 0)
            wait_idx(idx1, 0, 0)
            plsc.subcore_barrier()

            pltpu.sync_copy(xbuf.at[pl.ds(dst_off + r0, rows_per_tile)],
                            xs_hbm.at[r, row_slice, fcol])
            zero_rows_0()

            @pl.loop(0, n_rb)
            def _(b):
                pltpu.sync_copy(
                    rows_0, xbuf.at[pl.ds(src_off + r0 + b * CHUNK, CHUNK)])

            plsc.subcore_barrier()

    return prop_kernel(x_pad, src2, dst2, w2)


def _head_tc(x_pad, xs, w1, b1, w2, b2):
    n_pad, f = x_pad.shape
    order = xs.shape[0]
    hid = w1.shape[1]
    c_out = w2.shape[1]
    br = 1280

    def head_kernel(x_ref, xs_ref, w1_ref, b1_ref, w2_ref, b2_ref, o_ref):
        x = x_ref[...] + jnp.sum(xs_ref[...], axis=0)
        nrm = jnp.sqrt(jnp.sum(x * x, axis=1, keepdims=True))
        x = x / (1e-12 + nrm)
        h = jnp.dot(x, w1_ref[...], preferred_element_type=jnp.float32)
        h = h + b1_ref[...]
        h = jnp.maximum(h, 0.0)
        hn = jnp.sqrt(jnp.sum(h * h, axis=1, keepdims=True))
        h = h / (1e-12 + hn)
        o = jnp.dot(h, w2_ref[...], preferred_element_type=jnp.float32)
        o_ref[...] = o + b2_ref[...]

    return pl.pallas_call(
        head_kernel,
        grid=(n_pad // br,),
        in_specs=[
            pl.BlockSpec((br, f), lambda i: (i, 0)),
            pl.BlockSpec((order, br, f), lambda i: (0, i, 0)),
            pl.BlockSpec((f, hid), lambda i: (0, 0)),
            pl.BlockSpec((1, hid), lambda i: (0, 0)),
            pl.BlockSpec((hid, c_out), lambda i: (0, 0)),
            pl.BlockSpec((1, c_out), lambda i: (0, 0)),
        ],
        out_specs=pl.BlockSpec((br, c_out), lambda i: (i, 0)),
        out_shape=jax.ShapeDtypeStruct((n_pad, c_out), jnp.float32),
    )(x_pad, xs, w1, b1, w2, b2)


def kernel(X, edge_index, edge_weight, W1, b1, W2, b2):
    n, f = X.shape
    e = edge_weight.shape[0]
    n_pad = ((n + NS * L - 1) // (NS * L)) * (NS * L)
    step = NS * GRP * CHUNK
    e_pad = ((e + step - 1) // step) * step

    src = edge_index[0].astype(jnp.int32)
    dst = edge_index[1].astype(jnp.int32)
    w = edge_weight.astype(jnp.float32)
    if e_pad != e:
        pad = e_pad - e
        src = jnp.concatenate([src, jnp.zeros((pad,), jnp.int32)])
        dst = jnp.concatenate([dst, jnp.zeros((pad,), jnp.int32)])
        w = jnp.concatenate([w, jnp.zeros((pad,), jnp.float32)])
    src2 = src.reshape(e_pad // CHUNK, CHUNK)
    dst2 = dst.reshape(e_pad // CHUNK, CHUNK)
    w2 = w.reshape(e_pad // CHUNK, CHUNK)
    x_pad = X if n_pad == n else jnp.pad(X, ((0, n_pad - n), (0, 0)))

    xs = _prop_sc(n_pad, f, e_pad, x_pad, src2, dst2, w2)
    out = _head_tc(x_pad, xs, W1, b1.reshape(1, -1), W2, b2.reshape(1, -1))
    return out[:n]
